# initial kernel scaffold (unmeasured)
import jax
import jax.numpy as jnp
from jax import lax
from jax.experimental import pallas as pl
from jax.experimental.pallas import tpu as pltpu

N_DEV = 8


def kernel(x, w_mat, scale_x, scale_w):
    m_total, k_per = x.shape
    k_total, n = w_mat.shape
    m_per = m_total // N_DEV

    def body(x_ref, w_ref, sx_ref, sw_ref, out_ref, comm_ref, send_sems, recv_sems):
        my = lax.axis_index("i")

        barrier_sem = pltpu.get_barrier_semaphore()
        for p in range(N_DEV):
            pl.semaphore_signal(
                barrier_sem, inc=1,
                device_id=(p,), device_id_type=pl.DeviceIdType.MESH,
            )
        pl.semaphore_wait(barrier_sem, N_DEV)

        def send_desc(d):
            return pltpu.make_async_remote_copy(
                src_ref=x_ref.at[pl.ds(d * m_per, m_per), :],
                dst_ref=comm_ref.at[my],
                send_sem=send_sems.at[d],
                recv_sem=recv_sems.at[my],
                device_id=(d,),
                device_id_type=pl.DeviceIdType.MESH,
            )

        def recv_desc(s):
            return pltpu.make_async_remote_copy(
                src_ref=comm_ref.at[s],
                dst_ref=comm_ref.at[s],
                send_sem=send_sems.at[s],
                recv_sem=recv_sems.at[s],
                device_id=(s,),
                device_id_type=pl.DeviceIdType.MESH,
            )

        for d in range(N_DEV):
            send_desc(d).start()

        scale = sx_ref[0] * sw_ref[0]
        acc = jnp.zeros((m_per, n), jnp.float32)
        for s in range(N_DEV):
            recv_desc(s).wait_recv()
            acc = acc + jnp.dot(
                comm_ref[s],
                w_ref[pl.ds(s * k_per, k_per), :],
                preferred_element_type=jnp.float32,
            )
        out_ref[:, :] = acc * scale

        for d in range(N_DEV):
            send_desc(d).wait_send()

    return pl.pallas_call(
        body,
        out_shape=jax.ShapeDtypeStruct((m_per, n), jnp.float32),
        in_specs=[
            pl.BlockSpec(memory_space=pltpu.VMEM),
            pl.BlockSpec(memory_space=pltpu.VMEM),
            pl.BlockSpec(memory_space=pltpu.SMEM),
            pl.BlockSpec(memory_space=pltpu.SMEM),
        ],
        out_specs=pl.BlockSpec(memory_space=pltpu.VMEM),
        scratch_shapes=[
            pltpu.VMEM((N_DEV, m_per, k_per), x.dtype),
            pltpu.SemaphoreType.DMA((N_DEV,)),
            pltpu.SemaphoreType.DMA((N_DEV,)),
        ],
        compiler_params=pltpu.CompilerParams(collective_id=0),
    )(x, w_mat, scale_x, scale_w)


# baseline (device time: 95772 ns/iter reference)
import jax
import jax.numpy as jnp
from jax import lax
from jax.experimental import pallas as pl
from jax.experimental.pallas import tpu as pltpu

N_DEV = 8
N_TILE = 2048


def kernel(x, w_mat, scale_x, scale_w):
    m_total, k_per = x.shape
    k_total, n = w_mat.shape
    m_per = m_total // N_DEV
    n_tiles = n // N_TILE

    def body(x_ref, w_ref, sx_ref, sw_ref, out_ref,
             comm_ref, stage_ref, send_sems, recv_sems):
        ni = pl.program_id(0)
        s = pl.program_id(1)
        my = lax.axis_index("i")

        def send_desc(d):
            return pltpu.make_async_remote_copy(
                src_ref=stage_ref.at[d],
                dst_ref=comm_ref.at[my],
                send_sem=send_sems.at[d],
                recv_sem=recv_sems.at[my],
                device_id=(d,),
                device_id_type=pl.DeviceIdType.MESH,
            )

        def recv_desc(src):
            return pltpu.make_async_remote_copy(
                src_ref=comm_ref.at[src],
                dst_ref=comm_ref.at[src],
                send_sem=send_sems.at[src],
                recv_sem=recv_sems.at[src],
                device_id=(src,),
                device_id_type=pl.DeviceIdType.MESH,
            )

        @pl.when(jnp.logical_and(ni == 0, s == 0))
        def _():
            barrier_sem = pltpu.get_barrier_semaphore()
            for p in range(N_DEV):
                pl.semaphore_signal(
                    barrier_sem, inc=1,
                    device_id=(p,), device_id_type=pl.DeviceIdType.MESH,
                )
            pl.semaphore_wait(barrier_sem, N_DEV)
            for d in range(N_DEV):
                stage_ref[d] = x_ref[pl.ds(d * m_per, m_per), :].astype(
                    jnp.float8_e5m2
                )
            for d in range(N_DEV):
                send_desc(d).start()

        @pl.when(ni == 0)
        def _():
            recv_desc(s).wait_recv()

        part = jnp.dot(
            comm_ref[s],
            w_ref[...].astype(jnp.float8_e5m2),
            preferred_element_type=jnp.float32,
        )

        @pl.when(s == 0)
        def _():
            out_ref[...] = part

        @pl.when(s > 0)
        def _():
            out_ref[...] += part

        @pl.when(s == N_DEV - 1)
        def _():
            out_ref[...] *= sx_ref[0] * sw_ref[0]

        @pl.when(jnp.logical_and(ni == n_tiles - 1, s == N_DEV - 1))
        def _():
            for d in range(N_DEV):
                send_desc(d).wait_send()

    return pl.pallas_call(
        body,
        grid=(n_tiles, N_DEV),
        out_shape=jax.ShapeDtypeStruct((m_per, n), jnp.float32),
        in_specs=[
            pl.BlockSpec((m_total, k_per), lambda ni, s: (0, 0),
                         memory_space=pltpu.VMEM),
            pl.BlockSpec((k_per, N_TILE), lambda ni, s: (s, ni),
                         memory_space=pltpu.VMEM),
            pl.BlockSpec(memory_space=pltpu.SMEM),
            pl.BlockSpec(memory_space=pltpu.SMEM),
        ],
        out_specs=pl.BlockSpec((m_per, N_TILE), lambda ni, s: (0, ni),
                               memory_space=pltpu.VMEM),
        scratch_shapes=[
            pltpu.VMEM((N_DEV, m_per, k_per), jnp.float8_e5m2),
            pltpu.VMEM((N_DEV, m_per, k_per), jnp.float8_e5m2),
            pltpu.SemaphoreType.DMA((N_DEV,)),
            pltpu.SemaphoreType.DMA((N_DEV,)),
        ],
        compiler_params=pltpu.CompilerParams(
            collective_id=0,
            dimension_semantics=("arbitrary", "arbitrary"),
        ),
    )(x, w_mat, scale_x, scale_w)
